# baseline (speedup 1.0000x reference)
import jax
import jax.numpy as jnp
from jax.experimental import pallas as pl
from jax.experimental.pallas import tpu as pltpu
from jax.experimental.pallas import tpu_sc as plsc

F32 = jnp.float32
BF16 = jnp.bfloat16
_DN_LHS_T = (((0,), (0,)), ((), ()))


def _sc_gather(table, idx2d):
    n, c = table.shape
    e = idx2d.shape[1]
    w = 128
    assert e % w == 0
    mesh = plsc.VectorSubcoreMesh(core_axis_name="c", subcore_axis_name="s")

    @pl.kernel(out_type=jax.ShapeDtypeStruct((e, c), table.dtype), mesh=mesh)
    def gk(x_hbm, i_hbm, o_hbm):
        def body(i_vmem, o_vmem):
            pltpu.sync_copy(x_hbm.at[i_vmem.at[0]], o_vmem)

        pltpu.emit_pipeline(
            body,
            grid=(e // w,),
            in_specs=[pl.BlockSpec((1, w), lambda i: (0, i))],
            out_specs=[pl.BlockSpec((w, c), lambda i: (i, 0))],
            core_axis_name=("c", "s"),
            dimension_semantics=(pltpu.PARALLEL,),
        )(i_hbm, o_hbm)

    return gk(table, idx2d)


def _transpose_nf(Ft, interpret=False):
    C, N = Ft.shape

    def body(x_ref, o_ref):
        o_ref[...] = x_ref[...].T

    return pl.pallas_call(
        body,
        grid=(1,),
        in_specs=[pl.BlockSpec((C, N), lambda i: (0, 0))],
        out_specs=pl.BlockSpec((N, C), lambda i: (0, 0)),
        out_shape=jax.ShapeDtypeStruct((N, C), Ft.dtype),
        interpret=interpret,
    )(Ft)


def _colsum(y):
    ones = jnp.full((8, y.shape[0]), 1.0, dtype=F32)
    return jax.lax.dot_general(ones, y, (((1,), (0,)), ((), ())),
                               preferred_element_type=F32)


def _pass1a(PW2, WPW1h, EB, interpret=False):
    CW, E = PW2.shape
    CP = WPW1h.shape[1]

    def body(pw_ref, wp_ref, sp_ref, ssp_ref):
        i = pl.program_id(0)
        yp = jax.lax.dot_general(pw_ref[...].astype(BF16), wp_ref[...],
                                 _DN_LHS_T, preferred_element_type=F32)

        @pl.when(i == 0)
        def _():
            sp_ref[...] = jnp.zeros_like(sp_ref)
            ssp_ref[...] = jnp.zeros_like(ssp_ref)

        sp_ref[...] += _colsum(yp)
        ssp_ref[...] += _colsum(yp * yp)

    return pl.pallas_call(
        body,
        grid=(E // EB,),
        in_specs=[
            pl.BlockSpec((CW, EB), lambda i: (0, i)),
            pl.BlockSpec(WPW1h.shape, lambda i: (0, 0)),
        ],
        out_specs=[
            pl.BlockSpec((8, CP), lambda i: (0, 0)),
            pl.BlockSpec((8, CP), lambda i: (0, 0)),
        ],
        out_shape=[
            jax.ShapeDtypeStruct((8, CP), F32),
            jax.ShapeDtypeStruct((8, CP), F32),
        ],
        interpret=interpret,
    )(PW2, WPW1h)


def _pass1b(F, nknn, WS1h, WK1h, REP1h, T1, interpret=False):
    N, NIN = F.shape
    E = nknn.shape[0]
    K = E // N
    E1 = T1 * K
    CN = WS1h.shape[1]

    def body(f_ref, nk_ref, ws_ref, wk_ref, rep_ref, sn_ref, ssn_ref):
        i = pl.program_id(0)
        ys = jnp.dot(f_ref[...].astype(BF16), ws_ref[...],
                     preferred_element_type=F32)
        yk = jnp.dot(nk_ref[...].astype(BF16), wk_ref[...],
                     preferred_element_type=F32)
        yrep = jnp.dot(rep_ref[...], ys.astype(BF16),
                       preferred_element_type=F32)
        yn = yk + yrep

        @pl.when(i == 0)
        def _():
            sn_ref[...] = jnp.zeros_like(sn_ref)
            ssn_ref[...] = jnp.zeros_like(ssn_ref)

        sn_ref[...] += _colsum(yn)
        ssn_ref[...] += _colsum(yn * yn)

    return pl.pallas_call(
        body,
        grid=(N // T1,),
        in_specs=[
            pl.BlockSpec((T1, NIN), lambda i: (i, 0)),
            pl.BlockSpec((E1, NIN), lambda i: (i, 0)),
            pl.BlockSpec(WS1h.shape, lambda i: (0, 0)),
            pl.BlockSpec(WK1h.shape, lambda i: (0, 0)),
            pl.BlockSpec(REP1h.shape, lambda i: (0, 0)),
        ],
        out_specs=[
            pl.BlockSpec((8, CN), lambda i: (0, 0)),
            pl.BlockSpec((8, CN), lambda i: (0, 0)),
        ],
        out_shape=[
            jax.ShapeDtypeStruct((8, CN), F32),
            jax.ShapeDtypeStruct((8, CN), F32),
        ],
        interpret=interpret,
    )(F, nknn, WS1h, WK1h, REP1h)


def _pass2(F, nknn, PW2, ET2, REP2, EXP4, ws1f, wk1f, b1n, wpw1f, b1p,
           w2n, b2n, w2p, b2p, w2q, b2q, m1, n2, b2d, v2, b2w,
           wf, flt2, nflt, bias, T2, interpret=False):
    N, NIN = F.shape
    E = nknn.shape[0]
    K = E // N
    E2 = T2 * K
    CMIX = m1.shape[1]
    CO = n2.shape[1]
    CF = flt2.shape[1]
    CN = ws1f.shape[1]
    CP = wpw1f.shape[1]
    CWD = w2n.shape[0]
    CPU = w2p.shape[0]
    CW = PW2.shape[0]
    NET = ET2.shape[0]

    def body(f_ref, nk_ref, pw_ref, et_ref, rep_ref, exp_ref,
             ws_ref, wk_ref, b1n_ref, wp_ref, b1p_ref,
             w2n_ref, b2n_ref, w2p_ref, b2p_ref, w2q_ref, b2q_ref,
             m1_ref, n2_ref, b2d_ref, v2_ref, b2w_ref,
             wf_ref, flt2_ref, nflt_ref, bias_ref,
             ymix_ref, op_ref, smx_ref, ssmx_ref, sot_ref, ssot_ref):
        i = pl.program_id(0)
        hs = jnp.dot(f_ref[...], ws_ref[...], preferred_element_type=F32)
        hk = jnp.dot(nk_ref[...], wk_ref[...], preferred_element_type=F32)
        h = hk + jnp.dot(rep_ref[...], hs, preferred_element_type=F32) + b1n_ref[...]
        h = jnp.maximum(h, 0.0)
        p = jax.lax.dot_general(pw_ref[...], wp_ref[...], _DN_LHS_T,
                                preferred_element_type=F32) + b1p_ref[...]
        p = jnp.maximum(p, 0.0)
        nfeat_dy = jnp.dot(h[:, :CWD], w2n_ref[...], preferred_element_type=F32) + b2n_ref[...]
        w_og = jnp.dot(p[:, :CPU], w2p_ref[...], preferred_element_type=F32) + b2p_ref[...]
        w_plus = jnp.dot(p[:, CPU:2 * CPU], w2q_ref[...], preferred_element_type=F32) + b2q_ref[...]
        mix = w_og + nfeat_dy * w_plus
        ymix = jnp.dot(mix, m1_ref[...], preferred_element_type=F32)
        ymix_ref[...] = ymix
        pn = jnp.dot(h[:, CWD:], n2_ref[...], preferred_element_type=F32) + b2d_ref[...]
        pwo = jnp.dot(p[:, 2 * CPU:], v2_ref[...], preferred_element_type=F32) + b2w_ref[...]
        med = jnp.dot(pwo, wf_ref[...], preferred_element_type=F32)
        pn = pn * med
        ef = jnp.dot(pn, flt2_ref[...], preferred_element_type=F32)
        nf = jnp.dot(f_ref[...], nflt_ref[...], preferred_element_type=F32)
        smd = ef + jnp.dot(rep_ref[...], nf, preferred_element_type=F32)
        etexp = jax.lax.dot_general(et_ref[...], exp_ref[...], _DN_LHS_T,
                                    preferred_element_type=F32)
        s512 = smd * etexp
        edge = (s512[:, 0:CO] + s512[:, CO:2 * CO]
                + s512[:, 2 * CO:3 * CO] + s512[:, 3 * CO:4 * CO])
        om = jnp.max(edge.reshape(T2, K, CO), axis=1) + bias_ref[...]
        op_ref[...] = om

        @pl.when(i == 0)
        def _():
            smx_ref[...] = jnp.zeros_like(smx_ref)
            ssmx_ref[...] = jnp.zeros_like(ssmx_ref)
            sot_ref[...] = jnp.zeros_like(sot_ref)
            ssot_ref[...] = jnp.zeros_like(ssot_ref)

        smx_ref[...] += _colsum(ymix)
        ssmx_ref[...] += _colsum(ymix * ymix)
        sot_ref[...] += _colsum(om)
        ssot_ref[...] += _colsum(om * om)

    const = lambda a: pl.BlockSpec(a.shape, lambda i: tuple(0 for _ in a.shape))
    return pl.pallas_call(
        body,
        grid=(N // T2,),
        in_specs=[
            pl.BlockSpec((T2, NIN), lambda i: (i, 0)),
            pl.BlockSpec((E2, NIN), lambda i: (i, 0)),
            pl.BlockSpec((CW, E2), lambda i: (0, i)),
            pl.BlockSpec((NET, E2), lambda i: (0, i)),
            const(REP2), const(EXP4),
            const(ws1f), const(wk1f), const(b1n), const(wpw1f), const(b1p),
            const(w2n), const(b2n), const(w2p), const(b2p), const(w2q), const(b2q),
            const(m1), const(n2), const(b2d), const(v2), const(b2w),
            const(wf), const(flt2), const(nflt), const(bias),
        ],
        out_specs=[
            pl.BlockSpec((E2, CMIX), lambda i: (i, 0)),
            pl.BlockSpec((T2, CO), lambda i: (i, 0)),
            pl.BlockSpec((8, CMIX), lambda i: (0, 0)),
            pl.BlockSpec((8, CMIX), lambda i: (0, 0)),
            pl.BlockSpec((8, CO), lambda i: (0, 0)),
            pl.BlockSpec((8, CO), lambda i: (0, 0)),
        ],
        out_shape=[
            jax.ShapeDtypeStruct((E, CMIX), F32),
            jax.ShapeDtypeStruct((N, CO), F32),
            jax.ShapeDtypeStruct((8, CMIX), F32),
            jax.ShapeDtypeStruct((8, CMIX), F32),
            jax.ShapeDtypeStruct((8, CO), F32),
            jax.ShapeDtypeStruct((8, CO), F32),
        ],
        interpret=interpret,
    )(F, nknn, PW2, ET2, REP2, EXP4, ws1f, wk1f, b1n, wpw1f, b1p,
      w2n, b2n, w2p, b2p, w2q, b2q, m1, n2, b2d, v2, b2w,
      wf, flt2, nflt, bias)


def _pass3(ymix, outpre, ms, mt, w2m_om, b2m_c, osc, osh, T3, interpret=False):
    E, CMIX = ymix.shape
    N, CO = outpre.shape
    K = E // N
    E3 = T3 * K
    COUT = w2m_om.shape[0]

    def body(ym_ref, op_ref, ms_ref, mt_ref, w2m_ref, b2m_ref,
             os_ref, ot_ref, fw_ref, out_ref):
        z = jnp.maximum(ym_ref[...] * ms_ref[...] + mt_ref[...], 0.0)
        fw = jax.lax.dot_general(w2m_ref[...], z, (((1,), (1,)), ((), ())),
                                 preferred_element_type=F32)
        fw_ref[...] = fw + b2m_ref[...]
        out_ref[...] = jnp.maximum(op_ref[...] * os_ref[...] + ot_ref[...], 0.0)

    const = lambda a: pl.BlockSpec(a.shape, lambda i: tuple(0 for _ in a.shape))
    return pl.pallas_call(
        body,
        grid=(N // T3,),
        in_specs=[
            pl.BlockSpec((E3, CMIX), lambda i: (i, 0)),
            pl.BlockSpec((T3, CO), lambda i: (i, 0)),
            const(ms), const(mt), const(w2m_om), const(b2m_c),
            const(osc), const(osh),
        ],
        out_specs=[
            pl.BlockSpec((COUT, E3), lambda i: (0, i)),
            pl.BlockSpec((T3, CO), lambda i: (i, 0)),
        ],
        out_shape=[
            jax.ShapeDtypeStruct((COUT, E), F32),
            jax.ShapeDtypeStruct((N, CO), F32),
        ],
        interpret=interpret,
    )(ymix, outpre, ms, mt, w2m_om, b2m_c, osc, osh)


def _run(node_feature, pair_weight, nn_idx, etype, params,
         gather_fn=_sc_gather, interpret=False):
    p = params
    nin = node_feature.shape[1]
    n = node_feature.shape[2]
    k = nn_idx.shape[2]
    e = n * k
    net = etype.shape[1]
    nout = p["bias"].shape[0]

    Ft = node_feature[0, :, :, 0]
    PW2 = pair_weight[0].reshape(-1, e)
    ET2 = etype[0].reshape(-1, e)
    idx = nn_idx.reshape(1, e).astype(jnp.int32)
    F = _transpose_nf(Ft, interpret=interpret)
    nknn = gather_fn(F, idx)

    w1n = p["wdy_node"]["w1"].T
    w1d = p["node"]["w1"].T
    WS1 = jnp.concatenate([w1n[:nin], w1d[:nin]], axis=1)
    WK1 = jnp.concatenate([w1n[nin:], w1d[nin:]], axis=1)
    WPW1 = jnp.concatenate(
        [p["wdy_pure"]["w1"].T, p["wdy_plus"]["w1"].T, p["weight"]["w1"].T],
        axis=1)

    T1, T2 = 200, 80
    REP1 = (jnp.arange(T1 * k)[:, None] // k
            == jnp.arange(T1)[None, :]).astype(BF16)
    REP2 = (jnp.arange(T2 * k)[:, None] // k
            == jnp.arange(T2)[None, :]).astype(F32)
    EXP4 = (jnp.arange(net)[:, None]
            == (jnp.arange(net * nout)[None, :] // nout)).astype(F32)

    spa, sspa = _pass1a(PW2, WPW1.astype(BF16), EB=6400, interpret=interpret)
    sna, ssna = _pass1b(F, nknn, WS1.astype(BF16), WK1.astype(BF16),
                        REP1, T1=T1, interpret=interpret)
    sn, ssn, sp, ssp = sna[0], ssna[0], spa[0], sspa[0]

    mean_n = sn / e
    var_n = ssn / e - mean_n * mean_n
    g_n = jnp.concatenate([p["wdy_node"]["g"], p["node"]["g"]])
    bt_n = jnp.concatenate([p["wdy_node"]["bt"], p["node"]["bt"]])
    sc_n = jax.lax.rsqrt(var_n + 1e-5) * g_n
    WS1f = WS1 * sc_n[None, :]
    WK1f = WK1 * sc_n[None, :]
    B1n = (bt_n - mean_n * sc_n)[None, :]

    mean_p = sp / e
    var_p = ssp / e - mean_p * mean_p
    g_p = jnp.concatenate([p["wdy_pure"]["g"], p["wdy_plus"]["g"], p["weight"]["g"]])
    bt_p = jnp.concatenate([p["wdy_pure"]["bt"], p["wdy_plus"]["bt"], p["weight"]["bt"]])
    sc_p = jax.lax.rsqrt(var_p + 1e-5) * g_p
    WPW1f = WPW1 * sc_p[None, :]
    B1p = (bt_p - mean_p * sc_p)[None, :]

    W2n = p["wdy_node"]["w2"].T
    B2n = p["wdy_node"]["b2"][None, :]
    W2p = p["wdy_pure"]["w2"].T
    B2p = p["wdy_pure"]["b2"][None, :]
    W2q = p["wdy_plus"]["w2"].T
    B2q = p["wdy_plus"]["b2"][None, :]
    M1 = p["wdy_mix"]["w1"].T
    N2 = p["node"]["w2"].T
    B2d = p["node"]["b2"][None, :]
    V2 = p["weight"]["w2"].T
    B2w = p["weight"]["b2"][None, :]
    WF = p["wfilter_node"]
    FLT2 = p["filters2"].transpose(0, 2, 1).reshape(nout, nout * net)
    NFLT = p["nfilter"].reshape(nin, nout, net).transpose(0, 2, 1).reshape(nin, nout * net)
    BIAS = p["bias"][None, :]

    ymix, outpre, smx, ssmx, sot, ssot = _pass2(
        F, nknn, PW2, ET2, REP2, EXP4, WS1f, WK1f, B1n, WPW1f, B1p,
        W2n, B2n, W2p, B2p, W2q, B2q, M1, N2, B2d, V2, B2w,
        WF, FLT2, NFLT, BIAS, T2=T2, interpret=interpret)

    mean_m = smx[0] / e
    var_m = ssmx[0] / e - mean_m * mean_m
    ms = jax.lax.rsqrt(var_m + 1e-5) * p["wdy_mix"]["g"]
    MS = ms[None, :]
    MT = (p["wdy_mix"]["bt"] - mean_m * ms)[None, :]
    W2m_om = p["wdy_mix"]["w2"]
    B2m_c = p["wdy_mix"]["b2"][:, None]

    mean_o = sot[0] / n
    var_o = ssot[0] / n - mean_o * mean_o
    osv = jax.lax.rsqrt(var_o + 1e-5) * p["bn_g"]
    OS = osv[None, :]
    OT = (p["bn_b"] - mean_o * osv)[None, :]

    fwcm, outf = _pass3(ymix, outpre, MS, MT, W2m_om, B2m_c, OS, OT, T3=T2,
                        interpret=interpret)

    final_weight = fwcm.reshape(-1, n, k)[None]
    out = outf.T[None, :, :, None]
    return (out, final_weight)


def kernel(node_feature, pair_weight, nn_idx, etype, params):
    return _run(node_feature, pair_weight, nn_idx, etype, params)


# speedup vs baseline: 1.4843x; 1.4843x over previous
import jax
import jax.numpy as jnp
from jax.experimental import pallas as pl
from jax.experimental.pallas import tpu as pltpu
from jax.experimental.pallas import tpu_sc as plsc

F32 = jnp.float32
BF16 = jnp.bfloat16


def _sc_gather(table, idx2d):
    n, c = table.shape
    e = idx2d.shape[1]
    w = 128
    assert e % w == 0
    mesh = plsc.VectorSubcoreMesh(core_axis_name="c", subcore_axis_name="s")

    @pl.kernel(out_type=jax.ShapeDtypeStruct((e, c), table.dtype), mesh=mesh)
    def gk(x_hbm, i_hbm, o_hbm):
        def body(i_vmem, o_vmem):
            pltpu.sync_copy(x_hbm.at[i_vmem.at[0]], o_vmem)

        pltpu.emit_pipeline(
            body,
            grid=(e // w,),
            in_specs=[pl.BlockSpec((1, w), lambda i: (0, i))],
            out_specs=[pl.BlockSpec((w, c), lambda i: (i, 0))],
            core_axis_name=("c", "s"),
            dimension_semantics=(pltpu.PARALLEL,),
        )(i_hbm, o_hbm)

    return gk(table, idx2d)


def _colsum(y):
    ones = jnp.full((8, y.shape[0]), 1.0, dtype=F32)
    return jax.lax.dot_general(ones, y, (((1,), (0,)), ((), ())),
                               preferred_element_type=F32)


def _pass1a(PWe, EB, interpret=False):
    E, CW = PWe.shape

    def body(pw_ref, s_ref, mu_ref):
        i = pl.program_id(0)
        x = pw_ref[...]
        s = jax.lax.dot_general(x, x, (((0,), (0,)), ((), ())),
                                preferred_element_type=F32)

        @pl.when(i == 0)
        def _():
            s_ref[...] = jnp.zeros_like(s_ref)
            mu_ref[...] = jnp.zeros_like(mu_ref)

        s_ref[...] += s
        mu_ref[...] += _colsum(x)

    return pl.pallas_call(
        body,
        grid=(E // EB,),
        in_specs=[
            pl.BlockSpec((EB, CW), lambda i: (i, 0)),
        ],
        out_specs=[
            pl.BlockSpec((CW, CW), lambda i: (0, 0)),
            pl.BlockSpec((8, CW), lambda i: (0, 0)),
        ],
        out_shape=[
            jax.ShapeDtypeStruct((CW, CW), F32),
            jax.ShapeDtypeStruct((8, CW), F32),
        ],
        interpret=interpret,
    )(PWe)


def _pass1b(F, nknn, REP1T, T1, interpret=False):
    N, NIN = F.shape
    E = nknn.shape[0]
    K = E // N
    E1 = T1 * K
    _DN_RR = (((0,), (0,)), ((), ()))

    def body(f_ref, nk_ref, rt_ref, sss_ref, ssk_ref, skk_ref,
             muf_ref, muk_ref):
        i = pl.program_id(0)
        f = f_ref[...]
        nk = nk_ref[...]
        g = jnp.dot(rt_ref[...], nk, preferred_element_type=F32)
        s_ss = jax.lax.dot_general(f, f, _DN_RR, preferred_element_type=F32)
        s_sk = jax.lax.dot_general(f, g, _DN_RR, preferred_element_type=F32)
        s_kk = jax.lax.dot_general(nk, nk, _DN_RR, preferred_element_type=F32)

        @pl.when(i == 0)
        def _():
            sss_ref[...] = jnp.zeros_like(sss_ref)
            ssk_ref[...] = jnp.zeros_like(ssk_ref)
            skk_ref[...] = jnp.zeros_like(skk_ref)
            muf_ref[...] = jnp.zeros_like(muf_ref)
            muk_ref[...] = jnp.zeros_like(muk_ref)

        sss_ref[...] += s_ss
        ssk_ref[...] += s_sk
        skk_ref[...] += s_kk
        muf_ref[...] += _colsum(f)
        muk_ref[...] += _colsum(nk)

    cc = lambda shp: pl.BlockSpec(shp, lambda i: (0, 0))
    return pl.pallas_call(
        body,
        grid=(N // T1,),
        in_specs=[
            pl.BlockSpec((T1, NIN), lambda i: (i, 0)),
            pl.BlockSpec((E1, NIN), lambda i: (i, 0)),
            pl.BlockSpec(REP1T.shape, lambda i: (0, 0)),
        ],
        out_specs=[cc((NIN, NIN)), cc((NIN, NIN)), cc((NIN, NIN)),
                   cc((8, NIN)), cc((8, NIN))],
        out_shape=[
            jax.ShapeDtypeStruct((NIN, NIN), F32),
            jax.ShapeDtypeStruct((NIN, NIN), F32),
            jax.ShapeDtypeStruct((NIN, NIN), F32),
            jax.ShapeDtypeStruct((8, NIN), F32),
            jax.ShapeDtypeStruct((8, NIN), F32),
        ],
        interpret=interpret,
    )(F, nknn, REP1T)


def _pass2(F, nknn, PWe, ETe, ws1f, wk1f, b1n, wpw1f, b1p,
           w2n, b2n, w2p, b2p, w2q, b2q, m1, n2, b2d, v2, b2w,
           wf, flt2, nflt, bias, T2, interpret=False):
    N, NIN = F.shape
    E = nknn.shape[0]
    K = E // N
    E2 = T2 * K
    CMIX = m1.shape[1]
    CO = n2.shape[1]
    CN = ws1f.shape[1]
    CP = wpw1f.shape[1]
    CWD = w2n.shape[0]
    CPU = w2p.shape[0]
    CW = PWe.shape[1]
    NET = ETe.shape[1]

    def body(f_ref, nk_ref, pw_ref, et_ref,
             ws_ref, wk_ref, b1n_ref, wp_ref, b1p_ref,
             w2n_ref, b2n_ref, w2p_ref, b2p_ref, w2q_ref, b2q_ref,
             m1_ref, n2_ref, b2d_ref, v2_ref, b2w_ref,
             wf_ref, flt2_ref, nflt_ref, bias_ref,
             ymix_ref, op_ref, smx_ref, ssmx_ref, sot_ref, ssot_ref):
        i = pl.program_id(0)
        hs = jnp.dot(f_ref[...], ws_ref[...], preferred_element_type=F32)
        hk = jnp.dot(nk_ref[...], wk_ref[...], preferred_element_type=F32)
        h = ((hk.reshape(T2, K, CN) + hs[:, None, :]).reshape(E2, CN)
             + b1n_ref[...])
        h = jnp.maximum(h, 0.0)
        p = jnp.dot(pw_ref[...], wp_ref[...], preferred_element_type=F32) + b1p_ref[...]
        p = jnp.maximum(p, 0.0)
        nfeat_dy = jnp.dot(h[:, :CWD], w2n_ref[...], preferred_element_type=F32) + b2n_ref[...]
        w_og = jnp.dot(p[:, :CPU], w2p_ref[...], preferred_element_type=F32) + b2p_ref[...]
        w_plus = jnp.dot(p[:, CPU:2 * CPU], w2q_ref[...], preferred_element_type=F32) + b2q_ref[...]
        mix = w_og + nfeat_dy * w_plus
        ymix = jnp.dot(mix, m1_ref[...], preferred_element_type=F32)
        ymix_ref[...] = ymix.astype(BF16)
        pn = jnp.dot(h[:, CWD:], n2_ref[...], preferred_element_type=F32) + b2d_ref[...]
        pwo = jnp.dot(p[:, 2 * CPU:], v2_ref[...], preferred_element_type=F32) + b2w_ref[...]
        med = jnp.dot(pwo, wf_ref[...], preferred_element_type=F32)
        pn = pn * med
        ef = jnp.dot(pn, flt2_ref[...], preferred_element_type=F32)
        nf = jnp.dot(f_ref[...], nflt_ref[...], preferred_element_type=F32)
        CF = 4 * CO
        smd = (ef.reshape(T2, K, CF) + nf[:, None, :]).reshape(E2, CF)
        et = et_ref[...]
        edge = (smd[:, 0:CO] * et[:, 0:1]
                + smd[:, CO:2 * CO] * et[:, 1:2]
                + smd[:, 2 * CO:3 * CO] * et[:, 2:3]
                + smd[:, 3 * CO:4 * CO] * et[:, 3:4])
        om = jnp.max(edge.reshape(T2, K, CO), axis=1) + bias_ref[...]
        op_ref[...] = om

        @pl.when(i == 0)
        def _():
            smx_ref[...] = jnp.zeros_like(smx_ref)
            ssmx_ref[...] = jnp.zeros_like(ssmx_ref)
            sot_ref[...] = jnp.zeros_like(sot_ref)
            ssot_ref[...] = jnp.zeros_like(ssot_ref)

        smx_ref[...] += _colsum(ymix)
        ssmx_ref[...] += _colsum(ymix * ymix)
        sot_ref[...] += _colsum(om)
        ssot_ref[...] += _colsum(om * om)

    const = lambda a: pl.BlockSpec(a.shape, lambda i: tuple(0 for _ in a.shape))
    return pl.pallas_call(
        body,
        grid=(N // T2,),
        in_specs=[
            pl.BlockSpec((T2, NIN), lambda i: (i, 0)),
            pl.BlockSpec((E2, NIN), lambda i: (i, 0)),
            pl.BlockSpec((E2, CW), lambda i: (i, 0)),
            pl.BlockSpec((E2, NET), lambda i: (i, 0)),
            const(ws1f), const(wk1f), const(b1n), const(wpw1f), const(b1p),
            const(w2n), const(b2n), const(w2p), const(b2p), const(w2q), const(b2q),
            const(m1), const(n2), const(b2d), const(v2), const(b2w),
            const(wf), const(flt2), const(nflt), const(bias),
        ],
        out_specs=[
            pl.BlockSpec((E2, CMIX), lambda i: (i, 0)),
            pl.BlockSpec((T2, CO), lambda i: (i, 0)),
            pl.BlockSpec((8, CMIX), lambda i: (0, 0)),
            pl.BlockSpec((8, CMIX), lambda i: (0, 0)),
            pl.BlockSpec((8, CO), lambda i: (0, 0)),
            pl.BlockSpec((8, CO), lambda i: (0, 0)),
        ],
        out_shape=[
            jax.ShapeDtypeStruct((E, CMIX), BF16),
            jax.ShapeDtypeStruct((N, CO), F32),
            jax.ShapeDtypeStruct((8, CMIX), F32),
            jax.ShapeDtypeStruct((8, CMIX), F32),
            jax.ShapeDtypeStruct((8, CO), F32),
            jax.ShapeDtypeStruct((8, CO), F32),
        ],
        interpret=interpret,
    )(F, nknn, PWe, ETe, ws1f, wk1f, b1n, wpw1f, b1p,
      w2n, b2n, w2p, b2p, w2q, b2q, m1, n2, b2d, v2, b2w,
      wf, flt2, nflt, bias)


def _pass3(ymix, outpre, ms, mt, w2m, b2m, osc, osh, T3, interpret=False):
    E, CMIX = ymix.shape
    N, CO = outpre.shape
    K = E // N
    E3 = T3 * K

    def body(ym_ref, op_ref, ms_ref, mt_ref, w2m_ref, b2m_ref,
             os_ref, ot_ref, fw_ref, out_ref):
        z = jnp.maximum(ym_ref[...].astype(F32) * ms_ref[...] + mt_ref[...], 0.0)
        fw_ref[...] = jnp.dot(z, w2m_ref[...], preferred_element_type=F32) + b2m_ref[...]
        out_ref[...] = jnp.maximum(op_ref[...] * os_ref[...] + ot_ref[...], 0.0)

    const = lambda a: pl.BlockSpec(a.shape, lambda i: tuple(0 for _ in a.shape))
    return pl.pallas_call(
        body,
        grid=(N // T3,),
        in_specs=[
            pl.BlockSpec((E3, CMIX), lambda i: (i, 0)),
            pl.BlockSpec((T3, CO), lambda i: (i, 0)),
            const(ms), const(mt), const(w2m), const(b2m), const(osc), const(osh),
        ],
        out_specs=[
            pl.BlockSpec((E3, w2m.shape[1]), lambda i: (i, 0)),
            pl.BlockSpec((T3, CO), lambda i: (i, 0)),
        ],
        out_shape=[
            jax.ShapeDtypeStruct((E, w2m.shape[1]), F32),
            jax.ShapeDtypeStruct((N, CO), F32),
        ],
        interpret=interpret,
    )(ymix, outpre, ms, mt, w2m, b2m, osc, osh)


def _run(node_feature, pair_weight, nn_idx, etype, params,
         gather_fn=_sc_gather, interpret=False):
    p = params
    nin = node_feature.shape[1]
    n = node_feature.shape[2]
    k = nn_idx.shape[2]
    e = n * k
    net = etype.shape[1]
    nout = p["bias"].shape[0]

    F = node_feature[0, :, :, 0].T
    idx = nn_idx.reshape(1, e).astype(jnp.int32)
    PWe = pair_weight[0].transpose(1, 2, 0).reshape(e, -1)
    ETe = etype[0].transpose(1, 2, 0).reshape(e, -1)
    nknn = gather_fn(F, idx)

    w1n = p["wdy_node"]["w1"].T
    w1d = p["node"]["w1"].T
    WS1 = jnp.concatenate([w1n[:nin], w1d[:nin]], axis=1)
    WK1 = jnp.concatenate([w1n[nin:], w1d[nin:]], axis=1)
    WPW1 = jnp.concatenate(
        [p["wdy_pure"]["w1"].T, p["wdy_plus"]["w1"].T, p["weight"]["w1"].T],
        axis=1)

    T1, T2 = 200, 200
    REP1T = (jnp.arange(T1)[:, None]
             == jnp.arange(T1 * k)[None, :] // k).astype(F32)
    Spw, mupw = _pass1a(PWe, EB=6400, interpret=interpret)
    Sss, Ssk, Skk, muf, muk = _pass1b(F, nknn, REP1T, T1=T1,
                                      interpret=interpret)

    mean_n = (k * (muf[0] @ WS1) + muk[0] @ WK1) / e
    eyy_n = (k * jnp.sum(WS1 * (Sss @ WS1), axis=0)
             + 2.0 * jnp.sum(WS1 * (Ssk @ WK1), axis=0)
             + jnp.sum(WK1 * (Skk @ WK1), axis=0)) / e
    var_n = eyy_n - mean_n * mean_n
    g_n = jnp.concatenate([p["wdy_node"]["g"], p["node"]["g"]])
    bt_n = jnp.concatenate([p["wdy_node"]["bt"], p["node"]["bt"]])
    sc_n = jax.lax.rsqrt(var_n + 1e-5) * g_n
    WS1f = WS1 * sc_n[None, :]
    WK1f = WK1 * sc_n[None, :]
    B1n = (bt_n - mean_n * sc_n)[None, :]

    mean_p = (mupw[0] @ WPW1) / e
    var_p = jnp.sum(WPW1 * (Spw @ WPW1), axis=0) / e - mean_p * mean_p
    g_p = jnp.concatenate([p["wdy_pure"]["g"], p["wdy_plus"]["g"], p["weight"]["g"]])
    bt_p = jnp.concatenate([p["wdy_pure"]["bt"], p["wdy_plus"]["bt"], p["weight"]["bt"]])
    sc_p = jax.lax.rsqrt(var_p + 1e-5) * g_p
    WPW1f = WPW1 * sc_p[None, :]
    B1p = (bt_p - mean_p * sc_p)[None, :]

    W2n = p["wdy_node"]["w2"].T
    B2n = p["wdy_node"]["b2"][None, :]
    W2p = p["wdy_pure"]["w2"].T
    B2p = p["wdy_pure"]["b2"][None, :]
    W2q = p["wdy_plus"]["w2"].T
    B2q = p["wdy_plus"]["b2"][None, :]
    M1 = p["wdy_mix"]["w1"].T
    N2 = p["node"]["w2"].T
    B2d = p["node"]["b2"][None, :]
    V2 = p["weight"]["w2"].T
    B2w = p["weight"]["b2"][None, :]
    WF = p["wfilter_node"]
    FLT2 = p["filters2"].transpose(0, 2, 1).reshape(nout, nout * net)
    NFLT = p["nfilter"].reshape(nin, nout, net).transpose(0, 2, 1).reshape(nin, nout * net)
    BIAS = p["bias"][None, :]

    ymix, outpre, smx, ssmx, sot, ssot = _pass2(
        F, nknn, PWe, ETe, WS1f, WK1f, B1n, WPW1f, B1p,
        W2n, B2n, W2p, B2p, W2q, B2q, M1, N2, B2d, V2, B2w,
        WF, FLT2, NFLT, BIAS, T2=T2, interpret=interpret)

    mean_m = smx[0] / e
    var_m = ssmx[0] / e - mean_m * mean_m
    ms = jax.lax.rsqrt(var_m + 1e-5) * p["wdy_mix"]["g"]
    MS = ms[None, :]
    MT = (p["wdy_mix"]["bt"] - mean_m * ms)[None, :]
    W2m = p["wdy_mix"]["w2"].T
    B2m = p["wdy_mix"]["b2"][None, :]

    mean_o = sot[0] / n
    var_o = ssot[0] / n - mean_o * mean_o
    osv = jax.lax.rsqrt(var_o + 1e-5) * p["bn_g"]
    OS = osv[None, :]
    OT = (p["bn_b"] - mean_o * osv)[None, :]

    fw, outf = _pass3(ymix, outpre, MS, MT, W2m, B2m, OS, OT, T3=T2,
                      interpret=interpret)

    final_weight = fw.reshape(n, k, -1).transpose(2, 0, 1)[None]
    out = outf.T[None, :, :, None]
    return (out, final_weight)


def kernel(node_feature, pair_weight, nn_idx, etype, params):
    return _run(node_feature, pair_weight, nn_idx, etype, params)
